# baseline (speedup 1.0000x reference)
import functools

import jax
import jax.numpy as jnp
from jax import lax
from jax.experimental import pallas as pl
from jax.experimental.pallas import tpu as pltpu
from jax.experimental.pallas import tpu_sc as plsc

BATCH = 16384
HIST = 200
EMBED = 32
VOCAB = 1000000

NUM_CORES = 2
NUM_SUBCORES = 16
NUM_WORKERS = NUM_CORES * NUM_SUBCORES
BI = 128
NBB = BATCH // BI
BB_PER_W = NBB // NUM_WORKERS
EB = 4
EI = 8
NB = 4
GROUPS = HIST * BB_PER_W


def _build_gather_add():
  mesh = plsc.VectorSubcoreMesh(core_axis_name="c", subcore_axis_name="s")

  @functools.partial(
      pl.kernel,
      mesh=mesh,
      out_type=jax.ShapeDtypeStruct((HIST, EB, NBB, EI, BI), jnp.float32),
      compiler_params=pltpu.CompilerParams(
          needs_layout_passes=False, use_tc_tiling_on_sc=False),
      scratch_types=[
          pltpu.VMEM((HIST, BI), jnp.float32),
          pltpu.VMEM((NB, BI), jnp.float32),
          pltpu.VMEM((NB, 2, BI), jnp.float32),
          pltpu.VMEM((NB, BI), jnp.int32),
          pltpu.VMEM((NB, BI, EMBED), jnp.float32),
          pltpu.VMEM((NB, EB, EI, BI), jnp.float32),
          pltpu.SemaphoreType.DMA((NB,)),
          pltpu.SemaphoreType.DMA((NB,)),
          pltpu.SemaphoreType.DMA((NB,)),
      ],
  )
  def gather_add(table_hbm, x5_hbm, out_hbm, stage_v, nmean_v, io_v, idx_v,
                 rows_v, tr_v, io_sem, gather_sem, out_sem):

    wid = lax.axis_index("s") * NUM_CORES + lax.axis_index("c")
    bb0 = wid * BB_PER_W

    for k in range(BB_PER_W):
      pltpu.sync_copy(x5_hbm.at[:, bb0 + k, 1, :], stage_v)

      def accum(l, accs):
        return tuple(accs[j] + stage_v[l, pl.ds(j * 16, 16)]
                     for j in range(BI // 16))

      accs = lax.fori_loop(
          0, HIST, accum,
          tuple(jnp.zeros((16,), jnp.float32) for _ in range(BI // 16)))
      for j in range(BI // 16):
        nmean_v[k, pl.ds(j * 16, 16)] = accs[j] * (-1.0 / HIST)

    def issue_io(g):
      pltpu.async_copy(x5_hbm.at[g // NB, bb0 + (g % NB)], io_v.at[g % NB],
                       io_sem.at[g % NB])

    def wait_io(g):
      pltpu.make_async_copy(x5_hbm.at[g // NB, bb0 + (g % NB)],
                            io_v.at[g % NB], io_sem.at[g % NB]).wait()

    for b in range(NB):
      issue_io(b)
    wait_io(0)
    for j in range(BI // 16):
      idx_v[0, pl.ds(j * 16, 16)] = io_v[0, 0, pl.ds(j * 16, 16)].astype(
          jnp.int32)
    pltpu.async_copy(table_hbm.at[idx_v.at[0]], rows_v.at[0],
                     gather_sem.at[0])

    lane16 = lax.iota(jnp.int32, 16)

    def pipe(g, carry):
      p = g % NB
      pn = (g + 1) % NB

      @pl.when(g < GROUPS - 1)
      def _():
        wait_io(g + 1)
        for j in range(BI // 16):
          idx_v[pn, pl.ds(j * 16, 16)] = io_v[pn, 0,
                                              pl.ds(j * 16, 16)].astype(
                                                  jnp.int32)

        @pl.when(g + 1 >= NB)
        def _():
          pltpu.make_async_copy(tr_v.at[pn],
                                out_hbm.at[(g + 1 - NB) // NB, :,
                                           bb0 + pn],
                                out_sem.at[pn]).wait()

        pltpu.async_copy(table_hbm.at[idx_v.at[pn]], rows_v.at[pn],
                         gather_sem.at[pn])

      pltpu.make_async_copy(table_hbm.at[idx_v.at[p]], rows_v.at[p],
                            gather_sem.at[p]).wait()

      avs = tuple(io_v[p, 1, pl.ds(j * 16, 16)] + nmean_v[p,
                                                          pl.ds(j * 16, 16)]
                  for j in range(BI // 16))

      @pl.when(g + NB < GROUPS)
      def _():
        issue_io(g + NB)

      rowvs = tuple(lane16 + j * 16 for j in range(BI // 16))
      rp = rows_v.at[p]
      tpf = tr_v.at[p]

      @plsc.parallel_loop(0, EB * EI, unroll=4)
      def col_write(col):
        ev = (jnp.full((16,), col, jnp.int32) + lane16) & (EMBED - 1)
        ebv = ev >> 3
        eiv = ev & (EI - 1)
        for j in range(BI // 16):
          vals = plsc.load_gather(rp, [rowvs[j], ev]) + avs[j]
          plsc.store_scatter(tpf, [ebv, eiv, rowvs[j]], vals)

      pltpu.async_copy(tr_v.at[p], out_hbm.at[g // NB, :, bb0 + p],
                       out_sem.at[p])
      return carry

    lax.fori_loop(0, GROUPS, pipe, 0)

    for b in range(NB):
      pltpu.make_async_copy(tr_v.at[b],
                            out_hbm.at[(GROUPS - NB + b) // NB, :, bb0 + b],
                            out_sem.at[b]).wait()

  return gather_add


_gather_add = _build_gather_add()


def kernel(x, table):
  x5 = x.transpose(1, 0, 2).reshape(HIST, NBB, BI, 2).transpose(0, 1, 3, 2)
  tbl = jax.lax.optimization_barrier(table.reshape(VOCAB // 4, EMBED * 4))
  out5 = _gather_add(tbl.reshape(VOCAB, EMBED), x5)
  return out5.transpose(2, 4, 0, 1, 3).reshape(BATCH, HIST, EMBED)


# speedup vs baseline: 1.0689x; 1.0689x over previous
import functools

import jax
import jax.numpy as jnp
from jax import lax
from jax.experimental import pallas as pl
from jax.experimental.pallas import tpu as pltpu
from jax.experimental.pallas import tpu_sc as plsc

BATCH = 16384
HIST = 200
EMBED = 32
VOCAB = 1000000

NUM_CORES = 2
NUM_SUBCORES = 16
NUM_WORKERS = NUM_CORES * NUM_SUBCORES
BI = 128
NBB = BATCH // BI
BB_PER_W = NBB // NUM_WORKERS
EB = 4
EI = 8
NB = 4
GROUPS = HIST * BB_PER_W


def _build_gather_add():
  mesh = plsc.VectorSubcoreMesh(core_axis_name="c", subcore_axis_name="s")

  @functools.partial(
      pl.kernel,
      mesh=mesh,
      out_type=jax.ShapeDtypeStruct((HIST, EB, NBB, EI, BI), jnp.float32),
      compiler_params=pltpu.CompilerParams(
          needs_layout_passes=False, use_tc_tiling_on_sc=False),
      scratch_types=[
          pltpu.VMEM((HIST, BI), jnp.float32),
          pltpu.VMEM((NB, BI), jnp.float32),
          pltpu.VMEM((NB, 2, BI), jnp.float32),
          pltpu.VMEM((NB, BI), jnp.int32),
          pltpu.VMEM((NB, BI, EMBED), jnp.float32),
          pltpu.VMEM((NB, BI * (EMBED + 1)), jnp.float32),
          pltpu.VMEM((NB, EB, EI, BI), jnp.float32),
          pltpu.SemaphoreType.DMA((NB,)),
          pltpu.SemaphoreType.DMA((NB,)),
          pltpu.SemaphoreType.DMA((NB,)),
      ],
  )
  def gather_add(table_hbm, x5_hbm, out_hbm, stage_v, nmean_v, io_v, idx_v,
                 rows_v, rowsp_v, tr_v, io_sem, gather_sem, out_sem):

    wid = lax.axis_index("s") * NUM_CORES + lax.axis_index("c")
    bb0 = wid * BB_PER_W

    for k in range(BB_PER_W):
      pltpu.sync_copy(x5_hbm.at[:, bb0 + k, 1, :], stage_v)

      def accum(l, accs):
        return tuple(accs[j] + stage_v[l, pl.ds(j * 16, 16)]
                     for j in range(BI // 16))

      accs = lax.fori_loop(
          0, HIST, accum,
          tuple(jnp.zeros((16,), jnp.float32) for _ in range(BI // 16)))
      for j in range(BI // 16):
        nmean_v[k, pl.ds(j * 16, 16)] = accs[j] * (-1.0 / HIST)

    def issue_io(g):
      pltpu.async_copy(x5_hbm.at[g // NB, bb0 + (g % NB)], io_v.at[g % NB],
                       io_sem.at[g % NB])

    def wait_io(g):
      pltpu.make_async_copy(x5_hbm.at[g // NB, bb0 + (g % NB)],
                            io_v.at[g % NB], io_sem.at[g % NB]).wait()

    for b in range(NB):
      issue_io(b)
    wait_io(0)
    for j in range(BI // 16):
      idx_v[0, pl.ds(j * 16, 16)] = io_v[0, 0, pl.ds(j * 16, 16)].astype(
          jnp.int32) * 4
    pltpu.async_copy(table_hbm.at[idx_v.at[0]],
                     rows_v.at[0], gather_sem.at[0])

    lane16 = lax.iota(jnp.int32, 16)

    def pipe(g, carry):
      p = g % NB
      pn = (g + 1) % NB

      @pl.when(g < GROUPS - 1)
      def _():
        wait_io(g + 1)
        for j in range(BI // 16):
          idx_v[pn, pl.ds(j * 16, 16)] = io_v[pn, 0,
                                              pl.ds(j * 16, 16)].astype(
                                                  jnp.int32) * 4

        @pl.when(g + 1 >= NB)
        def _():
          pltpu.make_async_copy(tr_v.at[pn],
                                out_hbm.at[(g + 1 - NB) // NB, :,
                                           bb0 + pn],
                                out_sem.at[pn]).wait()

        pltpu.async_copy(table_hbm.at[idx_v.at[pn]],
                         rows_v.at[pn], gather_sem.at[pn])

      pltpu.make_async_copy(table_hbm.at[idx_v.at[p]],
                            rows_v.at[p], gather_sem.at[p]).wait()

      avs = tuple(io_v[p, 1, pl.ds(j * 16, 16)] + nmean_v[p,
                                                          pl.ds(j * 16, 16)]
                  for j in range(BI // 16))

      @pl.when(g + NB < GROUPS)
      def _():
        issue_io(g + NB)

      rp = rows_v.at[p]
      rpp = rowsp_v.at[p]

      @plsc.parallel_loop(0, BI, unroll=8)
      def repitch(r):
        rpp[pl.ds(r * (EMBED + 1), 16)] = rp[r, pl.ds(0, 16)]
        rpp[pl.ds(r * (EMBED + 1) + 16, 16)] = rp[r, pl.ds(16, 16)]

      rowvs = tuple(lane16 * (EMBED + 1) + (j * 16 * (EMBED + 1))
                    for j in range(BI // 16))
      tpf = tr_v.at[p]

      @plsc.parallel_loop(0, EB * EI, unroll=4)
      def col_write(col):
        colv = jnp.full((16,), col, jnp.int32)
        eb = col // EI
        ei = col % EI
        for j in range(BI // 16):
          vals = plsc.load_gather(rpp, [rowvs[j] + colv]) + avs[j]
          tpf[eb, ei, pl.ds(j * 16, 16)] = vals

      pltpu.async_copy(tr_v.at[p], out_hbm.at[g // NB, :, bb0 + p],
                       out_sem.at[p])
      return carry

    lax.fori_loop(0, GROUPS, pipe, 0)

    for b in range(NB):
      pltpu.make_async_copy(tr_v.at[b],
                            out_hbm.at[(GROUPS - NB + b) // NB, :, bb0 + b],
                            out_sem.at[b]).wait()

  return gather_add


_gather_add = _build_gather_add()


def kernel(x, table):
  x5 = x.transpose(1, 0, 2).reshape(HIST, NBB, BI, 2).transpose(0, 1, 3, 2)
  tblp = jnp.pad(table, ((0, 0), (0, 128 - EMBED))).reshape(4 * VOCAB, EMBED)
  out5 = _gather_add(tblp, x5)
  return out5.transpose(2, 4, 0, 1, 3).reshape(BATCH, HIST, EMBED)


# speedup vs baseline: 1.0750x; 1.0057x over previous
import functools

import jax
import jax.numpy as jnp
from jax import lax
from jax.experimental import pallas as pl
from jax.experimental.pallas import tpu as pltpu
from jax.experimental.pallas import tpu_sc as plsc

BATCH = 16384
HIST = 200
EMBED = 32
VOCAB = 1000000

NUM_CORES = 2
NUM_SUBCORES = 16
NUM_WORKERS = NUM_CORES * NUM_SUBCORES
BI = 128
NBB = BATCH // BI
BB_PER_W = NBB // NUM_WORKERS
EB = 4
EI = 8
NB = 4
GROUPS = HIST * BB_PER_W


def _build_neg_mean():
  mesh = plsc.VectorSubcoreMesh(core_axis_name="c", subcore_axis_name="s")

  @functools.partial(
      pl.kernel,
      mesh=mesh,
      out_type=jax.ShapeDtypeStruct((NBB, BI), jnp.float32),
      compiler_params=pltpu.CompilerParams(
          needs_layout_passes=False, use_tc_tiling_on_sc=False),
      scratch_types=[
          pltpu.VMEM((HIST, BI), jnp.float32),
          pltpu.VMEM((BB_PER_W, BI), jnp.float32),
      ],
  )
  def neg_mean(x5_hbm, nm_hbm, stage_v, nmean_v):
    wid = lax.axis_index("s") * NUM_CORES + lax.axis_index("c")
    bb0 = wid * BB_PER_W

    for k in range(BB_PER_W):
      pltpu.sync_copy(x5_hbm.at[:, bb0 + k, 1, :], stage_v)

      def accum(l, accs):
        return tuple(accs[j] + stage_v[l, pl.ds(j * 16, 16)]
                     for j in range(BI // 16))

      accs = lax.fori_loop(
          0, HIST, accum,
          tuple(jnp.zeros((16,), jnp.float32) for _ in range(BI // 16)))
      for j in range(BI // 16):
        nmean_v[k, pl.ds(j * 16, 16)] = accs[j] * (-1.0 / HIST)

    pltpu.sync_copy(nmean_v, nm_hbm.at[pl.ds(bb0, BB_PER_W)])

  return neg_mean


def _build_gather_add():
  mesh = plsc.VectorSubcoreMesh(core_axis_name="c", subcore_axis_name="s")

  @functools.partial(
      pl.kernel,
      mesh=mesh,
      out_type=jax.ShapeDtypeStruct((HIST, EB, NBB, EI, BI), jnp.float32),
      compiler_params=pltpu.CompilerParams(
          needs_layout_passes=False, use_tc_tiling_on_sc=False),
      scratch_types=[
          pltpu.VMEM((NB, BI), jnp.float32),
          pltpu.VMEM((NB, 2, BI), jnp.float32),
          pltpu.VMEM((NB, BI), jnp.int32),
          pltpu.VMEM((NB, BI, EMBED), jnp.float32),
          pltpu.VMEM((NB, BI * (EMBED + 1)), jnp.float32),
          pltpu.VMEM((NB, EB, EI, BI), jnp.float32),
          pltpu.SemaphoreType.DMA((NB,)),
          pltpu.SemaphoreType.DMA((NB,)),
          pltpu.SemaphoreType.DMA((NB,)),
      ],
  )
  def gather_add(table_hbm, x5_hbm, nm_hbm, out_hbm, nmean_v, io_v, idx_v,
                 rows_v, rowsp_v, tr_v, io_sem, gather_sem, out_sem):

    wid = lax.axis_index("s") * NUM_CORES + lax.axis_index("c")
    bb0 = wid * BB_PER_W

    pltpu.sync_copy(nm_hbm.at[pl.ds(bb0, BB_PER_W)], nmean_v)

    def issue_io(g):
      pltpu.async_copy(x5_hbm.at[g // NB, bb0 + (g % NB)], io_v.at[g % NB],
                       io_sem.at[g % NB])

    def wait_io(g):
      pltpu.make_async_copy(x5_hbm.at[g // NB, bb0 + (g % NB)],
                            io_v.at[g % NB], io_sem.at[g % NB]).wait()

    for b in range(NB):
      issue_io(b)
    wait_io(0)
    for j in range(BI // 16):
      idx_v[0, pl.ds(j * 16, 16)] = io_v[0, 0, pl.ds(j * 16, 16)].astype(
          jnp.int32) * 4
    pltpu.async_copy(table_hbm.at[idx_v.at[0]],
                     rows_v.at[0], gather_sem.at[0])

    lane16 = lax.iota(jnp.int32, 16)

    def pipe(g, carry):
      p = g % NB
      pn = (g + 1) % NB

      @pl.when(g < GROUPS - 1)
      def _():
        wait_io(g + 1)
        for j in range(BI // 16):
          idx_v[pn, pl.ds(j * 16, 16)] = io_v[pn, 0,
                                              pl.ds(j * 16, 16)].astype(
                                                  jnp.int32) * 4

        @pl.when(g + 1 >= NB)
        def _():
          pltpu.make_async_copy(tr_v.at[pn],
                                out_hbm.at[(g + 1 - NB) // NB, :,
                                           bb0 + pn],
                                out_sem.at[pn]).wait()

        pltpu.async_copy(table_hbm.at[idx_v.at[pn]],
                         rows_v.at[pn], gather_sem.at[pn])

      pltpu.make_async_copy(table_hbm.at[idx_v.at[p]],
                            rows_v.at[p], gather_sem.at[p]).wait()

      avs = tuple(io_v[p, 1, pl.ds(j * 16, 16)] + nmean_v[p,
                                                          pl.ds(j * 16, 16)]
                  for j in range(BI // 16))

      @pl.when(g + NB < GROUPS)
      def _():
        issue_io(g + NB)

      rp = rows_v.at[p]
      rpp = rowsp_v.at[p]

      @plsc.parallel_loop(0, BI, unroll=8)
      def repitch(r):
        rpp[pl.ds(r * (EMBED + 1), 16)] = rp[r, pl.ds(0, 16)]
        rpp[pl.ds(r * (EMBED + 1) + 16, 16)] = rp[r, pl.ds(16, 16)]

      rowvs = tuple(lane16 * (EMBED + 1) + (j * 16 * (EMBED + 1))
                    for j in range(BI // 16))
      tpf = tr_v.at[p]

      @plsc.parallel_loop(0, EB * EI, unroll=4)
      def col_write(col):
        colv = jnp.full((16,), col, jnp.int32)
        eb = col // EI
        ei = col % EI
        for j in range(BI // 16):
          vals = plsc.load_gather(rpp, [rowvs[j] + colv]) + avs[j]
          tpf[eb, ei, pl.ds(j * 16, 16)] = vals

      pltpu.async_copy(tr_v.at[p], out_hbm.at[g // NB, :, bb0 + p],
                       out_sem.at[p])
      return carry

    lax.fori_loop(0, GROUPS, pipe, 0)

    for b in range(NB):
      pltpu.make_async_copy(tr_v.at[b],
                            out_hbm.at[(GROUPS - NB + b) // NB, :, bb0 + b],
                            out_sem.at[b]).wait()

  return gather_add


_neg_mean = _build_neg_mean()
_gather_add = _build_gather_add()


def kernel(x, table):
  x5 = x.transpose(1, 0, 2).reshape(HIST, NBB, BI, 2).transpose(0, 1, 3, 2)
  tblp = jnp.pad(table, ((0, 0), (0, 128 - EMBED))).reshape(4 * VOCAB, EMBED)
  nm = _neg_mean(x5)
  out5 = _gather_add(tblp, x5, nm)
  return out5.transpose(2, 4, 0, 1, 3).reshape(BATCH, HIST, EMBED)


# speedup vs baseline: 1.0770x; 1.0019x over previous
import functools

import jax
import jax.numpy as jnp
from jax import lax
from jax.experimental import pallas as pl
from jax.experimental.pallas import tpu as pltpu
from jax.experimental.pallas import tpu_sc as plsc

BATCH = 16384
HIST = 200
EMBED = 32
VOCAB = 1000000

NUM_CORES = 2
NUM_SUBCORES = 16
NUM_WORKERS = NUM_CORES * NUM_SUBCORES
BI = 128
NBB = BATCH // BI
BB_PER_W = NBB // NUM_WORKERS
EB = 4
EI = 8
NB = 4
GROUPS = HIST * BB_PER_W


def _build_neg_mean():
  mesh = plsc.VectorSubcoreMesh(core_axis_name="c", subcore_axis_name="s")

  @functools.partial(
      pl.kernel,
      mesh=mesh,
      out_type=jax.ShapeDtypeStruct((NBB, BI), jnp.float32),
      compiler_params=pltpu.CompilerParams(
          needs_layout_passes=False, use_tc_tiling_on_sc=False),
      scratch_types=[
          pltpu.VMEM((HIST, BI), jnp.float32),
          pltpu.VMEM((BB_PER_W, BI), jnp.float32),
      ],
  )
  def neg_mean(x5_hbm, nm_hbm, stage_v, nmean_v):
    wid = lax.axis_index("s") * NUM_CORES + lax.axis_index("c")
    bb0 = wid * BB_PER_W

    for k in range(BB_PER_W):
      pltpu.sync_copy(x5_hbm.at[:, bb0 + k, 1, :], stage_v)

      def accum(l, accs):
        return tuple(accs[j] + stage_v[l, pl.ds(j * 16, 16)]
                     for j in range(BI // 16))

      accs = lax.fori_loop(
          0, HIST, accum,
          tuple(jnp.zeros((16,), jnp.float32) for _ in range(BI // 16)))
      for j in range(BI // 16):
        nmean_v[k, pl.ds(j * 16, 16)] = accs[j] * (-1.0 / HIST)

    pltpu.sync_copy(nmean_v, nm_hbm.at[pl.ds(bb0, BB_PER_W)])

  return neg_mean


def _build_gather_add():
  mesh = plsc.VectorSubcoreMesh(core_axis_name="c", subcore_axis_name="s")

  @functools.partial(
      pl.kernel,
      mesh=mesh,
      out_type=jax.ShapeDtypeStruct((HIST, EB, NBB, EI, BI), jnp.float32),
      compiler_params=pltpu.CompilerParams(
          needs_layout_passes=False, use_tc_tiling_on_sc=False),
      scratch_types=[
          pltpu.VMEM((NB, BI), jnp.float32),
          pltpu.VMEM((NB, 2, BI), jnp.float32),
          pltpu.VMEM((NB, BI), jnp.int32),
          pltpu.VMEM((NB, BI, EMBED), jnp.float32),
          pltpu.VMEM((NB, BI * (EMBED + 1)), jnp.float32),
          pltpu.VMEM((NB, EB, EI, BI), jnp.float32),
          pltpu.SemaphoreType.DMA((NB,)),
          pltpu.SemaphoreType.DMA((NB,)),
          pltpu.SemaphoreType.DMA((NB,)),
      ],
  )
  def gather_add(table_hbm, x5_hbm, nm_hbm, out_hbm, nmean_v, io_v, idx_v,
                 rows_v, rowsp_v, tr_v, io_sem, gather_sem, out_sem):

    wid = lax.axis_index("s") * NUM_CORES + lax.axis_index("c")
    bb0 = wid * BB_PER_W

    pltpu.sync_copy(nm_hbm.at[pl.ds(bb0, BB_PER_W)], nmean_v)

    def issue_io(g):
      pltpu.async_copy(x5_hbm.at[g // NB, bb0 + (g % NB)], io_v.at[g % NB],
                       io_sem.at[g % NB])

    def wait_io(g):
      pltpu.make_async_copy(x5_hbm.at[g // NB, bb0 + (g % NB)],
                            io_v.at[g % NB], io_sem.at[g % NB]).wait()

    for b in range(NB):
      issue_io(b)
    wait_io(0)
    for j in range(BI // 16):
      idx_v[0, pl.ds(j * 16, 16)] = io_v[0, 0, pl.ds(j * 16, 16)].astype(
          jnp.int32) * 4
    pltpu.async_copy(table_hbm.at[idx_v.at[0]],
                     rows_v.at[0], gather_sem.at[0])

    lane16 = lax.iota(jnp.int32, 16)

    def pipe(g, carry):
      p = g % NB
      pn = (g + 1) % NB

      @pl.when(g < GROUPS - 1)
      def _():
        wait_io(g + 1)
        for j in range(BI // 16):
          idx_v[pn, pl.ds(j * 16, 16)] = io_v[pn, 0,
                                              pl.ds(j * 16, 16)].astype(
                                                  jnp.int32) * 4

        @pl.when(g + 1 >= NB)
        def _():
          pltpu.make_async_copy(tr_v.at[pn],
                                out_hbm.at[(g + 1 - NB) // NB, :,
                                           bb0 + pn],
                                out_sem.at[pn]).wait()

        pltpu.async_copy(table_hbm.at[idx_v.at[pn]],
                         rows_v.at[pn], gather_sem.at[pn])

      pltpu.make_async_copy(table_hbm.at[idx_v.at[p]],
                            rows_v.at[p], gather_sem.at[p]).wait()

      avs = tuple(io_v[p, 1, pl.ds(j * 16, 16)] + nmean_v[p,
                                                          pl.ds(j * 16, 16)]
                  for j in range(BI // 16))

      @pl.when(g + NB < GROUPS)
      def _():
        issue_io(g + NB)

      rp = rows_v.at[p]
      rpp = rowsp_v.at[p]

      @plsc.parallel_loop(0, BI, unroll=16)
      def repitch(r):
        rpp[pl.ds(r * (EMBED + 1), 16)] = rp[r, pl.ds(0, 16)]
        rpp[pl.ds(r * (EMBED + 1) + 16, 16)] = rp[r, pl.ds(16, 16)]

      rowvs = tuple(lane16 * (EMBED + 1) + (j * 16 * (EMBED + 1))
                    for j in range(BI // 16))
      tpf = tr_v.at[p]

      @plsc.parallel_loop(0, EB * EI, unroll=8)
      def col_write(col):
        colv = jnp.full((16,), col, jnp.int32)
        eb = col // EI
        ei = col % EI
        for j in range(BI // 16):
          vals = plsc.load_gather(rpp, [rowvs[j] + colv]) + avs[j]
          tpf[eb, ei, pl.ds(j * 16, 16)] = vals

      pltpu.async_copy(tr_v.at[p], out_hbm.at[g // NB, :, bb0 + p],
                       out_sem.at[p])
      return carry

    lax.fori_loop(0, GROUPS, pipe, 0)

    for b in range(NB):
      pltpu.make_async_copy(tr_v.at[b],
                            out_hbm.at[(GROUPS - NB + b) // NB, :, bb0 + b],
                            out_sem.at[b]).wait()

  return gather_add


_neg_mean = _build_neg_mean()
_gather_add = _build_gather_add()


def kernel(x, table):
  x5 = x.transpose(1, 0, 2).reshape(HIST, NBB, BI, 2).transpose(0, 1, 3, 2)
  tblp = jnp.pad(table, ((0, 0), (0, 128 - EMBED))).reshape(4 * VOCAB, EMBED)
  nm = _neg_mean(x5)
  out5 = _gather_add(tblp, x5, nm)
  return out5.transpose(2, 4, 0, 1, 3).reshape(BATCH, HIST, EMBED)
